# Initial kernel scaffold; baseline (speedup 1.0000x reference)
import functools

import jax
import jax.numpy as jnp
from jax import lax
from jax.experimental import pallas as pl
from jax.experimental.pallas import tpu as pltpu
from jax.experimental.pallas import tpu_sc as plsc

N = 50000
G = 512
N_PAD = 50176
DUMMY = N
E = 1_600_000
CHUNK = 128
NCORES, NSUB = 2, 16
NW = NCORES * NSUB
BF = 23
OUTER = 17
PER_W = BF * OUTER
E_PAD = NW * PER_W * CHUNK
RPT = N_PAD // NSUB
BLK = 512
NBLK = N_PAD // BLK

_MESH = dict(core_axis_name="c", subcore_axis_name="s")


def _make_sc_propagate(C):
    @functools.partial(
        pl.kernel,
        out_type=jax.ShapeDtypeStruct((NCORES, N_PAD, C), jnp.float32),
        mesh=plsc.VectorSubcoreMesh(**_MESH),
        scratch_types=[
            pltpu.VMEM((BF, CHUNK), jnp.int32),
            pltpu.VMEM((BF, CHUNK), jnp.int32),
            pltpu.VMEM((BF, CHUNK, C), jnp.float32),
            pltpu.VMEM_SHARED((N_PAD, C), jnp.float32),
            pltpu.SemaphoreType.DMA,
            pltpu.SemaphoreType.DMA,
        ],
    )
    def kern(y_hbm, src_hbm, dst_hbm, zinit_hbm, out_hbm,
             src_v, dst_v, rows_v, z_sh, gsem, ssem):
        c = lax.axis_index("c")
        s = lax.axis_index("s")
        w = c * NSUB + s
        pltpu.sync_copy(zinit_hbm.at[pl.ds(s * RPT, RPT)],
                        z_sh.at[pl.ds(s * RPT, RPT)])
        plsc.subcore_barrier()

        def group(i, carry):
            base = w * PER_W + i * BF
            pltpu.sync_copy(src_hbm.at[pl.ds(base, BF)], src_v)
            pltpu.sync_copy(dst_hbm.at[pl.ds(base, BF)], dst_v)
            gathers = [
                pltpu.async_copy(y_hbm.at[src_v.at[j]], rows_v.at[j], gsem)
                for j in range(BF)
            ]
            for d in gathers:
                d.wait()
            scatters = [
                pltpu.async_copy(rows_v.at[j], z_sh.at[dst_v.at[j]], ssem,
                                 add=True)
                for j in range(BF)
            ]
            for d in scatters:
                d.wait()
            return carry

        lax.fori_loop(0, OUTER, group, 0)
        plsc.subcore_barrier()
        pltpu.sync_copy(z_sh.at[pl.ds(s * RPT, RPT)],
                        out_hbm.at[c, pl.ds(s * RPT, RPT)])

    return kern


@functools.partial(
    pl.kernel,
    out_type=jax.ShapeDtypeStruct((NCORES, N_PAD, 1), jnp.float32),
    mesh=plsc.VectorSubcoreMesh(**_MESH),
    scratch_types=[
        pltpu.VMEM((BF, CHUNK), jnp.int32),
        pltpu.VMEM((CHUNK, 1), jnp.float32),
        pltpu.VMEM_SHARED((N_PAD, 1), jnp.float32),
        pltpu.SemaphoreType.DMA,
    ],
)
def _sc_degree(dst_hbm, ones_hbm, zinit_hbm, out_hbm,
               dst_v, ones_v, z_sh, ssem):
    c = lax.axis_index("c")
    s = lax.axis_index("s")
    w = c * NSUB + s
    pltpu.sync_copy(zinit_hbm.at[pl.ds(s * RPT, RPT)],
                    z_sh.at[pl.ds(s * RPT, RPT)])
    pltpu.sync_copy(ones_hbm, ones_v)
    plsc.subcore_barrier()

    def group(i, carry):
        base = w * PER_W + i * BF
        pltpu.sync_copy(dst_hbm.at[pl.ds(base, BF)], dst_v)
        scatters = [
            pltpu.async_copy(ones_v, z_sh.at[dst_v.at[j]], ssem, add=True)
            for j in range(BF)
        ]
        for d in scatters:
            d.wait()
        return carry

    lax.fori_loop(0, OUTER, group, 0)
    plsc.subcore_barrier()
    pltpu.sync_copy(z_sh.at[pl.ds(s * RPT, RPT)],
                    out_hbm.at[c, pl.ds(s * RPT, RPT)])


_sc_prop4 = _make_sc_propagate(4)
_sc_prop32 = _make_sc_propagate(32)


def _tc_prep(degp, x_pad):
    def body(degp_ref, x_ref, dis_ref, y1_ref):
        deg = degp_ref[0] + degp_ref[1] + 1.0
        dis = lax.rsqrt(deg)
        dis_ref[...] = dis
        y1_ref[...] = x_ref[...] * dis

    return pl.pallas_call(
        body,
        grid=(NBLK,),
        in_specs=[
            pl.BlockSpec((NCORES, BLK, 1), lambda i: (0, i, 0)),
            pl.BlockSpec((BLK, 4), lambda i: (i, 0)),
        ],
        out_specs=[
            pl.BlockSpec((BLK, 1), lambda i: (i, 0)),
            pl.BlockSpec((BLK, 4), lambda i: (i, 0)),
        ],
        out_shape=[
            jax.ShapeDtypeStruct((N_PAD, 1), jnp.float32),
            jax.ShapeDtypeStruct((N_PAD, 4), jnp.float32),
        ],
    )(degp, x_pad)


def _tc_mid(z1p, y1, dis, W1, b1):
    def body(zp_ref, y1_ref, dis_ref, W_ref, b_ref, y2_ref):
        dis = dis_ref[...]
        p = (zp_ref[0] + zp_ref[1] + y1_ref[...]) * dis
        h = jnp.dot(p, W_ref[...], preferred_element_type=jnp.float32)
        h = jnp.maximum(h + b_ref[...], 0.0)
        rows = pl.program_id(0) * BLK + lax.broadcasted_iota(
            jnp.int32, (BLK, 1), 0)
        y2_ref[...] = jnp.where(rows < N, h * dis, 0.0)

    return pl.pallas_call(
        body,
        grid=(NBLK,),
        in_specs=[
            pl.BlockSpec((NCORES, BLK, 4), lambda i: (0, i, 0)),
            pl.BlockSpec((BLK, 4), lambda i: (i, 0)),
            pl.BlockSpec((BLK, 1), lambda i: (i, 0)),
            pl.BlockSpec((4, 32), lambda i: (0, 0)),
            pl.BlockSpec((1, 32), lambda i: (0, 0)),
        ],
        out_specs=pl.BlockSpec((BLK, 32), lambda i: (i, 0)),
        out_shape=jax.ShapeDtypeStruct((N_PAD, 32), jnp.float32),
    )(z1p, y1, dis, W1, b1)


def _tc_final(z2p, y2, dis, W2, b2, batch_pad, Wfc, bfc):
    def body(zp_ref, y2_ref, dis_ref, W_ref, b_ref, bt_ref, Wfc_ref, bfc_ref,
             out_ref, acc_ref):
        i = pl.program_id(0)
        dis = dis_ref[...]
        p = (zp_ref[0] + zp_ref[1] + y2_ref[...]) * dis
        h = jnp.dot(p, W_ref[...], preferred_element_type=jnp.float32)
        h = jnp.maximum(h + b_ref[...], 0.0)
        rows = i * BLK + lax.broadcasted_iota(jnp.int32, (BLK, 1), 0)
        valid = (rows < N).astype(jnp.float32)
        feat = jnp.concatenate([h * valid, valid], axis=1)
        onehot = (bt_ref[...] == lax.broadcasted_iota(
            jnp.int32, (BLK, G), 1)).astype(jnp.float32)
        contrib = lax.dot_general(
            onehot, feat, (((0,), (0,)), ((), ())),
            preferred_element_type=jnp.float32)

        @pl.when(i == 0)
        def _():
            acc_ref[...] = contrib

        @pl.when(i > 0)
        def _():
            acc_ref[...] = acc_ref[...] + contrib

        @pl.when(i == NBLK - 1)
        def _():
            ssum = acc_ref[:, :64]
            cnt = acc_ref[:, 64:65]
            g = ssum / jnp.maximum(cnt, 1.0)
            out_ref[...] = jnp.dot(
                g, Wfc_ref[...],
                preferred_element_type=jnp.float32) + bfc_ref[...]

    return pl.pallas_call(
        body,
        grid=(NBLK,),
        in_specs=[
            pl.BlockSpec((NCORES, BLK, 32), lambda i: (0, i, 0)),
            pl.BlockSpec((BLK, 32), lambda i: (i, 0)),
            pl.BlockSpec((BLK, 1), lambda i: (i, 0)),
            pl.BlockSpec((32, 64), lambda i: (0, 0)),
            pl.BlockSpec((1, 64), lambda i: (0, 0)),
            pl.BlockSpec((BLK, 1), lambda i: (i, 0)),
            pl.BlockSpec((64, 2), lambda i: (0, 0)),
            pl.BlockSpec((1, 2), lambda i: (0, 0)),
        ],
        out_specs=pl.BlockSpec((G, 2), lambda i: (0, 0)),
        out_shape=jax.ShapeDtypeStruct((G, 2), jnp.float32),
        scratch_shapes=[pltpu.VMEM((G, 65), jnp.float32)],
    )(z2p, y2, dis, W2, b2, batch_pad, Wfc, bfc)


def kernel(x, edge_index, batch, W1, b1, W2, b2, Wfc, bfc):
    ei = edge_index.astype(jnp.int32)
    pad = jnp.full((E_PAD - E,), DUMMY, jnp.int32)
    src = jnp.concatenate([ei[0], pad]).reshape(E_PAD // CHUNK, CHUNK)
    dst = jnp.concatenate([ei[1], pad]).reshape(E_PAD // CHUNK, CHUNK)
    x_pad = jnp.zeros((N_PAD, 4), jnp.float32).at[:N].set(x)
    batch_pad = jnp.zeros((N_PAD, 1), jnp.int32).at[:N, 0].set(
        batch.astype(jnp.int32))
    ones = jnp.ones((CHUNK, 1), jnp.float32)
    zin1 = jnp.zeros((N_PAD, 1), jnp.float32)
    zin4 = jnp.zeros((N_PAD, 4), jnp.float32)
    zin32 = jnp.zeros((N_PAD, 32), jnp.float32)

    degp = _sc_degree(dst, ones, zin1)
    dis, y1 = _tc_prep(degp, x_pad)
    z1p = _sc_prop4(y1, src, dst, zin4)
    y2 = _tc_mid(z1p, y1, dis, W1, b1.reshape(1, 32))
    z2p = _sc_prop32(y2, src, dst, zin32)
    return _tc_final(z2p, y2, dis, W2, b2.reshape(1, 64), batch_pad,
                     Wfc, bfc.reshape(1, 2))


# speedup vs baseline: 30.9515x; 30.9515x over previous
import functools

import jax
import jax.numpy as jnp
from jax import lax
from jax.experimental import pallas as pl
from jax.experimental.pallas import tpu as pltpu
from jax.experimental.pallas import tpu_sc as plsc

N = 50000
G = 512
N_PAD = 50176
DUMMY = N
E = 1_600_000
CHUNK = 128
NCORES, NSUB = 2, 16
NW = NCORES * NSUB
BF = 16
OUTER = 25
PER_W = BF * OUTER
E_PAD = NW * PER_W * CHUNK
RPT = N_PAD // NSUB
BLK = 512
NBLK = N_PAD // BLK

_MESH = dict(core_axis_name="c", subcore_axis_name="s")


@functools.partial(
    pl.kernel,
    out_type=jax.ShapeDtypeStruct((NCORES, N_PAD, 16), jnp.float32),
    mesh=plsc.VectorSubcoreMesh(**_MESH),
    scratch_types=[
        pltpu.VMEM((BF, CHUNK), jnp.int32),
        pltpu.VMEM((BF, CHUNK), jnp.int32),
        pltpu.VMEM((BF, CHUNK, 16), jnp.float32),
        pltpu.VMEM_SHARED((N_PAD, 16), jnp.float32),
        pltpu.SemaphoreType.DMA,
        pltpu.SemaphoreType.DMA,
    ],
    compiler_params=pltpu.CompilerParams(use_tc_tiling_on_sc=False),
)
def _sc_prop4(y_hbm, src_hbm, dst_hbm, zinit_hbm, out_hbm,
              src_v, dst_v, rows_v, z_sh, gsem, ssem):
    c = lax.axis_index("c")
    s = lax.axis_index("s")
    w = c * NSUB + s
    pltpu.sync_copy(zinit_hbm.at[pl.ds(s * RPT, RPT)],
                    z_sh.at[pl.ds(s * RPT, RPT)])
    plsc.subcore_barrier()

    def group(i, carry):
        base = w * PER_W + i * BF
        pltpu.sync_copy(src_hbm.at[pl.ds(base, BF)], src_v)
        pltpu.sync_copy(dst_hbm.at[pl.ds(base, BF)], dst_v)
        gathers = [
            pltpu.async_copy(y_hbm.at[src_v.at[j]], rows_v.at[j], gsem)
            for j in range(BF)
        ]
        for d in gathers:
            d.wait()
        scatters = [
            pltpu.async_copy(rows_v.at[j], z_sh.at[dst_v.at[j]], ssem,
                             add=True)
            for j in range(BF)
        ]
        for d in scatters:
            d.wait()
        return carry

    lax.fori_loop(0, OUTER, group, 0)
    plsc.subcore_barrier()
    pltpu.sync_copy(z_sh.at[pl.ds(s * RPT, RPT)],
                    out_hbm.at[c, pl.ds(s * RPT, RPT)])


PER_W2 = (E_PAD // CHUNK) // NSUB
OUTER2 = PER_W2 // BF

@functools.partial(
    pl.kernel,
    out_type=jax.ShapeDtypeStruct((NCORES, N_PAD, 16), jnp.float32),
    mesh=plsc.VectorSubcoreMesh(**_MESH),
    scratch_types=[
        pltpu.VMEM((BF, CHUNK), jnp.int32),
        pltpu.VMEM((BF, CHUNK), jnp.int32),
        pltpu.VMEM((BF, CHUNK, 16), jnp.float32),
        pltpu.VMEM_SHARED((N_PAD, 16), jnp.float32),
        pltpu.SemaphoreType.DMA,
        pltpu.SemaphoreType.DMA,
    ],
    compiler_params=pltpu.CompilerParams(use_tc_tiling_on_sc=False),
)
def _sc_prop_split(y_hbm, src_hbm, dst_hbm, zinit_hbm, out_hbm,
                   src_v, dst_v, rows_v, z_sh, gsem, ssem):
    c = lax.axis_index("c")
    s = lax.axis_index("s")
    ytab = y_hbm.at[c]
    pltpu.sync_copy(zinit_hbm.at[pl.ds(s * RPT, RPT)],
                    z_sh.at[pl.ds(s * RPT, RPT)])
    plsc.subcore_barrier()

    def group(i, carry):
        base = s * PER_W2 + i * BF
        pltpu.sync_copy(src_hbm.at[pl.ds(base, BF)], src_v)
        pltpu.sync_copy(dst_hbm.at[pl.ds(base, BF)], dst_v)
        gathers = [
            pltpu.async_copy(ytab.at[src_v.at[j]], rows_v.at[j], gsem)
            for j in range(BF)
        ]
        for d in gathers:
            d.wait()
        scatters = [
            pltpu.async_copy(rows_v.at[j], z_sh.at[dst_v.at[j]], ssem,
                             add=True)
            for j in range(BF)
        ]
        for d in scatters:
            d.wait()
        return carry

    lax.fori_loop(0, OUTER2, group, 0)
    plsc.subcore_barrier()
    pltpu.sync_copy(z_sh.at[pl.ds(s * RPT, RPT)],
                    out_hbm.at[c, pl.ds(s * RPT, RPT)])


@functools.partial(
    pl.kernel,
    out_type=jax.ShapeDtypeStruct((NCORES, N_PAD, 16), jnp.float32),
    mesh=plsc.VectorSubcoreMesh(**_MESH),
    scratch_types=[
        pltpu.VMEM((BF, CHUNK), jnp.int32),
        pltpu.VMEM((CHUNK, 16), jnp.float32),
        pltpu.VMEM_SHARED((N_PAD, 16), jnp.float32),
        pltpu.SemaphoreType.DMA,
    ],
    compiler_params=pltpu.CompilerParams(use_tc_tiling_on_sc=False),
)
def _sc_degree(dst_hbm, ones_hbm, zinit_hbm, out_hbm,
               dst_v, ones_v, z_sh, ssem):
    c = lax.axis_index("c")
    s = lax.axis_index("s")
    w = c * NSUB + s
    pltpu.sync_copy(zinit_hbm.at[pl.ds(s * RPT, RPT)],
                    z_sh.at[pl.ds(s * RPT, RPT)])
    pltpu.sync_copy(ones_hbm, ones_v)
    plsc.subcore_barrier()

    def group(i, carry):
        base = w * PER_W + i * BF
        pltpu.sync_copy(dst_hbm.at[pl.ds(base, BF)], dst_v)
        scatters = [
            pltpu.async_copy(ones_v, z_sh.at[dst_v.at[j]], ssem, add=True)
            for j in range(BF)
        ]
        for d in scatters:
            d.wait()
        return carry

    lax.fori_loop(0, OUTER, group, 0)
    plsc.subcore_barrier()
    pltpu.sync_copy(z_sh.at[pl.ds(s * RPT, RPT)],
                    out_hbm.at[c, pl.ds(s * RPT, RPT)])


def _tc_prep(degp, x_pad):
    def body(degp_ref, x_ref, dis_ref, y1_ref):
        deg = degp_ref[0, :, 0:1] + degp_ref[1, :, 0:1] + 1.0
        dis = lax.rsqrt(deg)
        dis_ref[...] = dis
        y1_ref[...] = jnp.concatenate(
            [x_ref[...] * dis, jnp.zeros((BLK, 12), jnp.float32)], axis=1)

    return pl.pallas_call(
        body,
        grid=(NBLK,),
        in_specs=[
            pl.BlockSpec((NCORES, BLK, 16), lambda i: (0, i, 0)),
            pl.BlockSpec((BLK, 4), lambda i: (i, 0)),
        ],
        out_specs=[
            pl.BlockSpec((BLK, 1), lambda i: (i, 0)),
            pl.BlockSpec((BLK, 16), lambda i: (i, 0)),
        ],
        out_shape=[
            jax.ShapeDtypeStruct((N_PAD, 1), jnp.float32),
            jax.ShapeDtypeStruct((N_PAD, 16), jnp.float32),
        ],
    )(degp, x_pad)


def _tc_mid(z1p, y1, dis, W1, b1):
    def body(zp_ref, y1_ref, dis_ref, W_ref, b_ref, y2_ref):
        dis = dis_ref[...]
        p = (zp_ref[0] + zp_ref[1] + y1_ref[...]) * dis
        h = jnp.dot(p, W_ref[...], preferred_element_type=jnp.float32)
        h = jnp.maximum(h + b_ref[...], 0.0)
        rows = pl.program_id(0) * BLK + lax.broadcasted_iota(
            jnp.int32, (BLK, 1), 0)
        y2 = jnp.where(rows < N, h * dis, 0.0)
        y2_ref[0] = y2[:, :16]
        y2_ref[1] = y2[:, 16:]

    return pl.pallas_call(
        body,
        grid=(NBLK,),
        in_specs=[
            pl.BlockSpec((NCORES, BLK, 16), lambda i: (0, i, 0)),
            pl.BlockSpec((BLK, 16), lambda i: (i, 0)),
            pl.BlockSpec((BLK, 1), lambda i: (i, 0)),
            pl.BlockSpec((16, 32), lambda i: (0, 0)),
            pl.BlockSpec((1, 32), lambda i: (0, 0)),
        ],
        out_specs=pl.BlockSpec((NCORES, BLK, 16), lambda i: (0, i, 0)),
        out_shape=jax.ShapeDtypeStruct((NCORES, N_PAD, 16), jnp.float32),
    )(z1p, y1, dis, W1, b1)


def _tc_final(z2p, y2, dis, W2, b2, batch_pad, Wfc, bfc):
    def body(zp_ref, y2_ref, dis_ref, W_ref, b_ref, bt_ref, Wfc_ref, bfc_ref,
             out_ref, acc_ref):
        i = pl.program_id(0)
        dis = dis_ref[...]
        z2 = jnp.concatenate([zp_ref[0], zp_ref[1]], axis=1)
        y2 = jnp.concatenate([y2_ref[0], y2_ref[1]], axis=1)
        p = (z2 + y2) * dis
        h = jnp.dot(p, W_ref[...], preferred_element_type=jnp.float32)
        h = jnp.maximum(h + b_ref[...], 0.0)
        rows = i * BLK + lax.broadcasted_iota(jnp.int32, (BLK, 1), 0)
        valid = (rows < N).astype(jnp.float32)
        feat = jnp.concatenate([h * valid, valid], axis=1)
        onehot = (bt_ref[...] == lax.broadcasted_iota(
            jnp.int32, (BLK, G), 1)).astype(jnp.float32)
        contrib = lax.dot_general(
            onehot, feat, (((0,), (0,)), ((), ())),
            preferred_element_type=jnp.float32)

        @pl.when(i == 0)
        def _():
            acc_ref[...] = contrib

        @pl.when(i > 0)
        def _():
            acc_ref[...] = acc_ref[...] + contrib

        @pl.when(i == NBLK - 1)
        def _():
            ssum = acc_ref[:, :64]
            cnt = acc_ref[:, 64:65]
            g = ssum / jnp.maximum(cnt, 1.0)
            out_ref[...] = jnp.dot(
                g, Wfc_ref[...],
                preferred_element_type=jnp.float32) + bfc_ref[...]

    return pl.pallas_call(
        body,
        grid=(NBLK,),
        in_specs=[
            pl.BlockSpec((NCORES, BLK, 16), lambda i: (0, i, 0)),
            pl.BlockSpec((NCORES, BLK, 16), lambda i: (0, i, 0)),
            pl.BlockSpec((BLK, 1), lambda i: (i, 0)),
            pl.BlockSpec((32, 64), lambda i: (0, 0)),
            pl.BlockSpec((1, 64), lambda i: (0, 0)),
            pl.BlockSpec((BLK, 1), lambda i: (i, 0)),
            pl.BlockSpec((64, 2), lambda i: (0, 0)),
            pl.BlockSpec((1, 2), lambda i: (0, 0)),
        ],
        out_specs=pl.BlockSpec((G, 2), lambda i: (0, 0)),
        out_shape=jax.ShapeDtypeStruct((G, 2), jnp.float32),
        scratch_shapes=[pltpu.VMEM((G, 65), jnp.float32)],
    )(z2p, y2, dis, W2, b2, batch_pad, Wfc, bfc)


def kernel(x, edge_index, batch, W1, b1, W2, b2, Wfc, bfc):
    ei = edge_index.astype(jnp.int32)
    pad = jnp.full((E_PAD - E,), DUMMY, jnp.int32)
    src = jnp.concatenate([ei[0], pad]).reshape(E_PAD // CHUNK, CHUNK)
    dst = jnp.concatenate([ei[1], pad]).reshape(E_PAD // CHUNK, CHUNK)
    x_pad = jnp.zeros((N_PAD, 4), jnp.float32).at[:N].set(x)
    batch_pad = jnp.zeros((N_PAD, 1), jnp.int32).at[:N, 0].set(
        batch.astype(jnp.int32))
    ones = jnp.ones((CHUNK, 16), jnp.float32)
    zin16 = jnp.zeros((N_PAD, 16), jnp.float32)
    W1p = jnp.zeros((16, 32), jnp.float32).at[:4].set(W1)

    degp = _sc_degree(dst, ones, zin16)
    dis, y1 = _tc_prep(degp, x_pad)
    z1p = _sc_prop4(y1, src, dst, zin16)
    y2h = _tc_mid(z1p, y1, dis, W1p, b1.reshape(1, 32))
    z2p = _sc_prop_split(y2h, src, dst, zin16)
    return _tc_final(z2p, y2h, dis, W2, b2.reshape(1, 64), batch_pad,
                     Wfc, bfc.reshape(1, 2))


# speedup vs baseline: 53.3206x; 1.7227x over previous
import functools

import jax
import jax.numpy as jnp
from jax import lax
from jax.experimental import pallas as pl
from jax.experimental.pallas import tpu as pltpu
from jax.experimental.pallas import tpu_sc as plsc

N = 50000
G = 512
N_PAD = 50176
E = 1_600_000
CHUNK = 128
NCORES, NSUB = 2, 16
NW = NCORES * NSUB
BF = 8
PER_W = 400
E_PAD = NW * PER_W * CHUNK
NGRP_A = PER_W // BF
PER_T = NW * PER_W // NSUB
NGRP_S = PER_T // BF
RPT = N_PAD // NSUB
BLK = 512
NBLK = N_PAD // BLK

_MESH = dict(core_axis_name="c", subcore_axis_name="s")
_SC_PARAMS = pltpu.CompilerParams(use_tc_tiling_on_sc=False)


def _edge_ring(ytab, src_hbm, dst_hbm, z_sh,
               sv0, sv1, dv0, dv1, r0, r1, gs0, gs1, ss0, ss1,
               base0, ngroups):
    def load(sv, dv, k):
        base = base0 + k * BF
        pltpu.sync_copy(src_hbm.at[pl.ds(base, BF)], sv)
        pltpu.sync_copy(dst_hbm.at[pl.ds(base, BF)], dv)

    def fire_g(sv, rv, sem):
        for j in range(BF):
            pltpu.async_copy(ytab.at[sv.at[j]], rv.at[j], sem)

    def drain_g(sv, rv, sem):
        for j in range(BF):
            pltpu.make_async_copy(ytab.at[sv.at[j]], rv.at[j], sem).wait()

    def fire_s(dv, rv, sem):
        for j in range(BF):
            pltpu.async_copy(rv.at[j], z_sh.at[dv.at[j]], sem, add=True)

    def drain_s(dv, rv, sem):
        for j in range(BF):
            pltpu.make_async_copy(rv.at[j], z_sh.at[dv.at[j]], sem).wait()

    load(sv0, dv0, 0)
    fire_g(sv0, r0, gs0)
    npair = ngroups // 2

    def pair(t, carry):
        @pl.when(t > 0)
        def _():
            drain_s(dv1, r1, ss1)

        load(sv1, dv1, 2 * t + 1)
        drain_g(sv0, r0, gs0)
        fire_s(dv0, r0, ss0)
        fire_g(sv1, r1, gs1)
        drain_s(dv0, r0, ss0)

        @pl.when(t + 1 < npair)
        def _():
            load(sv0, dv0, 2 * t + 2)
            fire_g(sv0, r0, gs0)

        drain_g(sv1, r1, gs1)
        fire_s(dv1, r1, ss1)
        return carry

    lax.fori_loop(0, npair, pair, 0)
    drain_s(dv1, r1, ss1)


_PROP_SCRATCH = [
    pltpu.VMEM((BF, CHUNK), jnp.int32),
    pltpu.VMEM((BF, CHUNK), jnp.int32),
    pltpu.VMEM((BF, CHUNK), jnp.int32),
    pltpu.VMEM((BF, CHUNK), jnp.int32),
    pltpu.VMEM((BF, CHUNK, 16), jnp.float32),
    pltpu.VMEM((BF, CHUNK, 16), jnp.float32),
    pltpu.VMEM_SHARED((N_PAD, 16), jnp.float32),
    pltpu.SemaphoreType.DMA,
    pltpu.SemaphoreType.DMA,
    pltpu.SemaphoreType.DMA,
    pltpu.SemaphoreType.DMA,
]


@functools.partial(
    pl.kernel,
    out_type=jax.ShapeDtypeStruct((NCORES, N_PAD, 16), jnp.float32),
    mesh=plsc.VectorSubcoreMesh(**_MESH),
    scratch_types=_PROP_SCRATCH,
    compiler_params=_SC_PARAMS,
)
def _sc_prop4(y_hbm, src_hbm, dst_hbm, zinit_hbm, out_hbm,
              sv0, sv1, dv0, dv1, r0, r1, z_sh, gs0, gs1, ss0, ss1):
    c = lax.axis_index("c")
    s = lax.axis_index("s")
    w = c * NSUB + s
    pltpu.sync_copy(zinit_hbm.at[pl.ds(s * RPT, RPT)],
                    z_sh.at[pl.ds(s * RPT, RPT)])
    plsc.subcore_barrier()
    _edge_ring(y_hbm, src_hbm, dst_hbm, z_sh,
               sv0, sv1, dv0, dv1, r0, r1, gs0, gs1, ss0, ss1,
               w * PER_W, NGRP_A)
    plsc.subcore_barrier()
    pltpu.sync_copy(z_sh.at[pl.ds(s * RPT, RPT)],
                    out_hbm.at[c, pl.ds(s * RPT, RPT)])


@functools.partial(
    pl.kernel,
    out_type=jax.ShapeDtypeStruct((NCORES, N_PAD, 16), jnp.float32),
    mesh=plsc.VectorSubcoreMesh(**_MESH),
    scratch_types=_PROP_SCRATCH,
    compiler_params=_SC_PARAMS,
)
def _sc_prop_split(y_hbm, src_hbm, dst_hbm, zinit_hbm, out_hbm,
                   sv0, sv1, dv0, dv1, r0, r1, z_sh, gs0, gs1, ss0, ss1):
    c = lax.axis_index("c")
    s = lax.axis_index("s")
    pltpu.sync_copy(zinit_hbm.at[pl.ds(s * RPT, RPT)],
                    z_sh.at[pl.ds(s * RPT, RPT)])
    plsc.subcore_barrier()
    _edge_ring(y_hbm.at[c], src_hbm, dst_hbm, z_sh,
               sv0, sv1, dv0, dv1, r0, r1, gs0, gs1, ss0, ss1,
               s * PER_T, NGRP_S)
    plsc.subcore_barrier()
    pltpu.sync_copy(z_sh.at[pl.ds(s * RPT, RPT)],
                    out_hbm.at[c, pl.ds(s * RPT, RPT)])


@functools.partial(
    pl.kernel,
    out_type=jax.ShapeDtypeStruct((NCORES, N_PAD, 16), jnp.float32),
    mesh=plsc.VectorSubcoreMesh(**_MESH),
    scratch_types=[
        pltpu.VMEM((BF, CHUNK), jnp.int32),
        pltpu.VMEM((BF, CHUNK), jnp.int32),
        pltpu.VMEM((CHUNK, 16), jnp.float32),
        pltpu.VMEM_SHARED((N_PAD, 16), jnp.float32),
        pltpu.SemaphoreType.DMA,
        pltpu.SemaphoreType.DMA,
    ],
    compiler_params=_SC_PARAMS,
)
def _sc_degree(dst_hbm, ones_hbm, zinit_hbm, out_hbm,
               dv0, dv1, ones_v, z_sh, ss0, ss1):
    c = lax.axis_index("c")
    s = lax.axis_index("s")
    w = c * NSUB + s
    base0 = w * PER_W
    pltpu.sync_copy(zinit_hbm.at[pl.ds(s * RPT, RPT)],
                    z_sh.at[pl.ds(s * RPT, RPT)])
    pltpu.sync_copy(ones_hbm, ones_v)
    plsc.subcore_barrier()

    def load(dv, k):
        pltpu.sync_copy(dst_hbm.at[pl.ds(base0 + k * BF, BF)], dv)

    def fire(dv, sem):
        for j in range(BF):
            pltpu.async_copy(ones_v, z_sh.at[dv.at[j]], sem, add=True)

    def drain(dv, sem):
        for j in range(BF):
            pltpu.make_async_copy(ones_v, z_sh.at[dv.at[j]], sem).wait()

    load(dv0, 0)
    fire(dv0, ss0)
    npair = NGRP_A // 2

    def pair(t, carry):
        @pl.when(t > 0)
        def _():
            drain(dv1, ss1)

        load(dv1, 2 * t + 1)
        fire(dv1, ss1)
        drain(dv0, ss0)

        @pl.when(t + 1 < npair)
        def _():
            load(dv0, 2 * t + 2)
            fire(dv0, ss0)

        return carry

    lax.fori_loop(0, npair, pair, 0)
    drain(dv1, ss1)
    plsc.subcore_barrier()
    pltpu.sync_copy(z_sh.at[pl.ds(s * RPT, RPT)],
                    out_hbm.at[c, pl.ds(s * RPT, RPT)])


def _tc_prep(degp, x_pad):
    def body(degp_ref, x_ref, dis_ref, y1_ref):
        deg = degp_ref[0, :, 0:1] + degp_ref[1, :, 0:1] + 1.0
        dis = lax.rsqrt(deg)
        dis_ref[...] = dis
        y1_ref[...] = jnp.concatenate(
            [x_ref[...] * dis, jnp.zeros((BLK, 12), jnp.float32)], axis=1)

    return pl.pallas_call(
        body,
        grid=(NBLK,),
        in_specs=[
            pl.BlockSpec((NCORES, BLK, 16), lambda i: (0, i, 0)),
            pl.BlockSpec((BLK, 4), lambda i: (i, 0)),
        ],
        out_specs=[
            pl.BlockSpec((BLK, 1), lambda i: (i, 0)),
            pl.BlockSpec((BLK, 16), lambda i: (i, 0)),
        ],
        out_shape=[
            jax.ShapeDtypeStruct((N_PAD, 1), jnp.float32),
            jax.ShapeDtypeStruct((N_PAD, 16), jnp.float32),
        ],
    )(degp, x_pad)


def _tc_mid(z1p, y1, dis, W1, b1):
    def body(zp_ref, y1_ref, dis_ref, W_ref, b_ref, y2_ref):
        dis = dis_ref[...]
        p = (zp_ref[0] + zp_ref[1] + y1_ref[...]) * dis
        h = jnp.dot(p, W_ref[...], preferred_element_type=jnp.float32)
        h = jnp.maximum(h + b_ref[...], 0.0)
        rows = pl.program_id(0) * BLK + lax.broadcasted_iota(
            jnp.int32, (BLK, 1), 0)
        y2 = jnp.where(rows < N, h * dis, 0.0)
        y2_ref[0] = y2[:, :16]
        y2_ref[1] = y2[:, 16:]

    return pl.pallas_call(
        body,
        grid=(NBLK,),
        in_specs=[
            pl.BlockSpec((NCORES, BLK, 16), lambda i: (0, i, 0)),
            pl.BlockSpec((BLK, 16), lambda i: (i, 0)),
            pl.BlockSpec((BLK, 1), lambda i: (i, 0)),
            pl.BlockSpec((16, 32), lambda i: (0, 0)),
            pl.BlockSpec((1, 32), lambda i: (0, 0)),
        ],
        out_specs=pl.BlockSpec((NCORES, BLK, 16), lambda i: (0, i, 0)),
        out_shape=jax.ShapeDtypeStruct((NCORES, N_PAD, 16), jnp.float32),
    )(z1p, y1, dis, W1, b1)


def _tc_final(z2p, y2, dis, W2, b2, batch_pad, Wfc, bfc):
    def body(zp_ref, y2_ref, dis_ref, W_ref, b_ref, bt_ref, Wfc_ref, bfc_ref,
             out_ref, acc_ref):
        i = pl.program_id(0)
        dis = dis_ref[...]
        z2 = jnp.concatenate([zp_ref[0], zp_ref[1]], axis=1)
        y2 = jnp.concatenate([y2_ref[0], y2_ref[1]], axis=1)
        p = (z2 + y2) * dis
        h = jnp.dot(p, W_ref[...], preferred_element_type=jnp.float32)
        h = jnp.maximum(h + b_ref[...], 0.0)
        rows = i * BLK + lax.broadcasted_iota(jnp.int32, (BLK, 1), 0)
        valid = (rows < N).astype(jnp.float32)
        feat = jnp.concatenate([h * valid, valid], axis=1)
        onehot = (bt_ref[...] == lax.broadcasted_iota(
            jnp.int32, (BLK, G), 1)).astype(jnp.float32)
        contrib = lax.dot_general(
            onehot, feat, (((0,), (0,)), ((), ())),
            preferred_element_type=jnp.float32)

        @pl.when(i == 0)
        def _():
            acc_ref[...] = contrib

        @pl.when(i > 0)
        def _():
            acc_ref[...] = acc_ref[...] + contrib

        @pl.when(i == NBLK - 1)
        def _():
            ssum = acc_ref[:, :64]
            cnt = acc_ref[:, 64:65]
            g = ssum / jnp.maximum(cnt, 1.0)
            out_ref[...] = jnp.dot(
                g, Wfc_ref[...],
                preferred_element_type=jnp.float32) + bfc_ref[...]

    return pl.pallas_call(
        body,
        grid=(NBLK,),
        in_specs=[
            pl.BlockSpec((NCORES, BLK, 16), lambda i: (0, i, 0)),
            pl.BlockSpec((NCORES, BLK, 16), lambda i: (0, i, 0)),
            pl.BlockSpec((BLK, 1), lambda i: (i, 0)),
            pl.BlockSpec((32, 64), lambda i: (0, 0)),
            pl.BlockSpec((1, 64), lambda i: (0, 0)),
            pl.BlockSpec((BLK, 1), lambda i: (i, 0)),
            pl.BlockSpec((64, 2), lambda i: (0, 0)),
            pl.BlockSpec((1, 2), lambda i: (0, 0)),
        ],
        out_specs=pl.BlockSpec((G, 2), lambda i: (0, 0)),
        out_shape=jax.ShapeDtypeStruct((G, 2), jnp.float32),
        scratch_shapes=[pltpu.VMEM((G, 65), jnp.float32)],
    )(z2p, y2, dis, W2, b2, batch_pad, Wfc, bfc)


def kernel(x, edge_index, batch, W1, b1, W2, b2, Wfc, bfc):
    ei = edge_index.astype(jnp.int32)
    pad = jnp.arange(E_PAD - E, dtype=jnp.int32) % (N_PAD - N) + N
    src = jnp.concatenate([ei[0], pad]).reshape(E_PAD // CHUNK, CHUNK)
    dst = jnp.concatenate([ei[1], pad]).reshape(E_PAD // CHUNK, CHUNK)
    x_pad = jnp.zeros((N_PAD, 4), jnp.float32).at[:N].set(x)
    batch_pad = jnp.zeros((N_PAD, 1), jnp.int32).at[:N, 0].set(
        batch.astype(jnp.int32))
    ones = jnp.ones((CHUNK, 16), jnp.float32)
    zin16 = jnp.zeros((N_PAD, 16), jnp.float32)
    W1p = jnp.zeros((16, 32), jnp.float32).at[:4].set(W1)

    degp = _sc_degree(dst, ones, zin16)
    dis, y1 = _tc_prep(degp, x_pad)
    z1p = _sc_prop4(y1, src, dst, zin16)
    y2h = _tc_mid(z1p, y1, dis, W1p, b1.reshape(1, 32))
    z2p = _sc_prop_split(y2h, src, dst, zin16)
    return _tc_final(z2p, y2h, dis, W2, b2.reshape(1, 64), batch_pad,
                     Wfc, bfc.reshape(1, 2))


# speedup vs baseline: 62.1092x; 1.1648x over previous
import functools

import jax
import jax.numpy as jnp
from jax import lax
from jax.experimental import pallas as pl
from jax.experimental.pallas import tpu as pltpu
from jax.experimental.pallas import tpu_sc as plsc

N = 50000
G = 512
N_PAD = 50176
E = 1_600_000
CHUNK = 128
NCORES, NSUB = 2, 16
NW = NCORES * NSUB
BF = 8
PER_W = 400
E_PAD = NW * PER_W * CHUNK
NGRP_A = PER_W // BF
PER_T = NW * PER_W // NSUB
NGRP_S = PER_T // BF
RPT = N_PAD // NSUB
BLK = 3584
NBLK = N_PAD // BLK

_MESH = dict(core_axis_name="c", subcore_axis_name="s")
_SC_PARAMS = pltpu.CompilerParams(use_tc_tiling_on_sc=False)


def _edge_ring(ytab, src_hbm, dst_hbm, z_sh,
               sv0, sv1, dv0, dv1, r0, r1, gs0, gs1, ss0, ss1,
               base0, ngroups):
    def load(sv, dv, k):
        base = base0 + k * BF
        pltpu.sync_copy(src_hbm.at[pl.ds(base, BF)], sv)
        pltpu.sync_copy(dst_hbm.at[pl.ds(base, BF)], dv)

    def fire_g(sv, rv, sem):
        for j in range(BF):
            pltpu.async_copy(ytab.at[sv.at[j]], rv.at[j], sem)

    def drain_g(sv, rv, sem):
        for j in range(BF):
            pltpu.make_async_copy(ytab.at[sv.at[j]], rv.at[j], sem).wait()

    def fire_s(dv, rv, sem):
        for j in range(BF):
            pltpu.async_copy(rv.at[j], z_sh.at[dv.at[j]], sem, add=True)

    def drain_s(dv, rv, sem):
        for j in range(BF):
            pltpu.make_async_copy(rv.at[j], z_sh.at[dv.at[j]], sem).wait()

    load(sv0, dv0, 0)
    fire_g(sv0, r0, gs0)
    npair = ngroups // 2

    def pair(t, carry):
        @pl.when(t > 0)
        def _():
            drain_s(dv1, r1, ss1)

        load(sv1, dv1, 2 * t + 1)
        drain_g(sv0, r0, gs0)
        fire_s(dv0, r0, ss0)
        fire_g(sv1, r1, gs1)
        drain_s(dv0, r0, ss0)

        @pl.when(t + 1 < npair)
        def _():
            load(sv0, dv0, 2 * t + 2)
            fire_g(sv0, r0, gs0)

        drain_g(sv1, r1, gs1)
        fire_s(dv1, r1, ss1)
        return carry

    lax.fori_loop(0, npair, pair, 0)
    drain_s(dv1, r1, ss1)


_PROP_SCRATCH = [
    pltpu.VMEM((BF, CHUNK), jnp.int32),
    pltpu.VMEM((BF, CHUNK), jnp.int32),
    pltpu.VMEM((BF, CHUNK), jnp.int32),
    pltpu.VMEM((BF, CHUNK), jnp.int32),
    pltpu.VMEM((BF, CHUNK, 16), jnp.float32),
    pltpu.VMEM((BF, CHUNK, 16), jnp.float32),
    pltpu.VMEM_SHARED((N_PAD, 16), jnp.float32),
    pltpu.SemaphoreType.DMA,
    pltpu.SemaphoreType.DMA,
    pltpu.SemaphoreType.DMA,
    pltpu.SemaphoreType.DMA,
]


@functools.partial(
    pl.kernel,
    out_type=jax.ShapeDtypeStruct((NCORES, N_PAD, 16), jnp.float32),
    mesh=plsc.VectorSubcoreMesh(**_MESH),
    scratch_types=_PROP_SCRATCH,
    compiler_params=_SC_PARAMS,
)
def _sc_prop4(y_hbm, src_hbm, dst_hbm, zinit_hbm, out_hbm,
              sv0, sv1, dv0, dv1, r0, r1, z_sh, gs0, gs1, ss0, ss1):
    c = lax.axis_index("c")
    s = lax.axis_index("s")
    w = c * NSUB + s
    pltpu.sync_copy(zinit_hbm.at[pl.ds(s * RPT, RPT)],
                    z_sh.at[pl.ds(s * RPT, RPT)])
    plsc.subcore_barrier()
    _edge_ring(y_hbm, src_hbm, dst_hbm, z_sh,
               sv0, sv1, dv0, dv1, r0, r1, gs0, gs1, ss0, ss1,
               w * PER_W, NGRP_A)
    plsc.subcore_barrier()
    pltpu.sync_copy(z_sh.at[pl.ds(s * RPT, RPT)],
                    out_hbm.at[c, pl.ds(s * RPT, RPT)])


@functools.partial(
    pl.kernel,
    out_type=jax.ShapeDtypeStruct((NCORES, N_PAD, 16), jnp.float32),
    mesh=plsc.VectorSubcoreMesh(**_MESH),
    scratch_types=_PROP_SCRATCH,
    compiler_params=_SC_PARAMS,
)
def _sc_prop_split(y_hbm, src_hbm, dst_hbm, zinit_hbm, out_hbm,
                   sv0, sv1, dv0, dv1, r0, r1, z_sh, gs0, gs1, ss0, ss1):
    c = lax.axis_index("c")
    s = lax.axis_index("s")
    pltpu.sync_copy(zinit_hbm.at[pl.ds(s * RPT, RPT)],
                    z_sh.at[pl.ds(s * RPT, RPT)])
    plsc.subcore_barrier()
    _edge_ring(y_hbm.at[c], src_hbm, dst_hbm, z_sh,
               sv0, sv1, dv0, dv1, r0, r1, gs0, gs1, ss0, ss1,
               s * PER_T, NGRP_S)
    plsc.subcore_barrier()
    pltpu.sync_copy(z_sh.at[pl.ds(s * RPT, RPT)],
                    out_hbm.at[c, pl.ds(s * RPT, RPT)])


@functools.partial(
    pl.kernel,
    out_type=jax.ShapeDtypeStruct((NCORES, N_PAD, 16), jnp.float32),
    mesh=plsc.VectorSubcoreMesh(**_MESH),
    scratch_types=[
        pltpu.VMEM((BF, CHUNK), jnp.int32),
        pltpu.VMEM((BF, CHUNK), jnp.int32),
        pltpu.VMEM((CHUNK, 16), jnp.float32),
        pltpu.VMEM_SHARED((N_PAD, 16), jnp.float32),
        pltpu.SemaphoreType.DMA,
        pltpu.SemaphoreType.DMA,
    ],
    compiler_params=_SC_PARAMS,
)
def _sc_degree(dst_hbm, ones_hbm, zinit_hbm, out_hbm,
               dv0, dv1, ones_v, z_sh, ss0, ss1):
    c = lax.axis_index("c")
    s = lax.axis_index("s")
    w = c * NSUB + s
    base0 = w * PER_W
    pltpu.sync_copy(zinit_hbm.at[pl.ds(s * RPT, RPT)],
                    z_sh.at[pl.ds(s * RPT, RPT)])
    pltpu.sync_copy(ones_hbm, ones_v)
    plsc.subcore_barrier()

    def load(dv, k):
        pltpu.sync_copy(dst_hbm.at[pl.ds(base0 + k * BF, BF)], dv)

    def fire(dv, sem):
        for j in range(BF):
            pltpu.async_copy(ones_v, z_sh.at[dv.at[j]], sem, add=True)

    def drain(dv, sem):
        for j in range(BF):
            pltpu.make_async_copy(ones_v, z_sh.at[dv.at[j]], sem).wait()

    load(dv0, 0)
    fire(dv0, ss0)
    npair = NGRP_A // 2

    def pair(t, carry):
        @pl.when(t > 0)
        def _():
            drain(dv1, ss1)

        load(dv1, 2 * t + 1)
        fire(dv1, ss1)
        drain(dv0, ss0)

        @pl.when(t + 1 < npair)
        def _():
            load(dv0, 2 * t + 2)
            fire(dv0, ss0)

        return carry

    lax.fori_loop(0, npair, pair, 0)
    drain(dv1, ss1)
    plsc.subcore_barrier()
    pltpu.sync_copy(z_sh.at[pl.ds(s * RPT, RPT)],
                    out_hbm.at[c, pl.ds(s * RPT, RPT)])


def _tc_prep(degp, x_pad):
    def body(degp_ref, x_ref, dis_ref, y1_ref):
        deg = degp_ref[0, :, 0:1] + degp_ref[1, :, 0:1] + 1.0
        dis = lax.rsqrt(deg)
        dis_ref[...] = dis
        y1_ref[...] = jnp.concatenate(
            [x_ref[...] * dis, jnp.zeros((BLK, 12), jnp.float32)], axis=1)

    return pl.pallas_call(
        body,
        grid=(NBLK,),
        in_specs=[
            pl.BlockSpec((NCORES, BLK, 16), lambda i: (0, i, 0)),
            pl.BlockSpec((BLK, 4), lambda i: (i, 0)),
        ],
        out_specs=[
            pl.BlockSpec((BLK, 1), lambda i: (i, 0)),
            pl.BlockSpec((BLK, 16), lambda i: (i, 0)),
        ],
        out_shape=[
            jax.ShapeDtypeStruct((N_PAD, 1), jnp.float32),
            jax.ShapeDtypeStruct((N_PAD, 16), jnp.float32),
        ],
    )(degp, x_pad)


def _tc_mid(z1p, y1, dis, W1, b1):
    def body(zp_ref, y1_ref, dis_ref, W_ref, b_ref, y2_ref):
        dis = dis_ref[...]
        p = (zp_ref[0] + zp_ref[1] + y1_ref[...]) * dis
        h = jnp.dot(p, W_ref[...], preferred_element_type=jnp.float32)
        h = jnp.maximum(h + b_ref[...], 0.0)
        rows = pl.program_id(0) * BLK + lax.broadcasted_iota(
            jnp.int32, (BLK, 1), 0)
        y2 = jnp.where(rows < N, h * dis, 0.0)
        y2_ref[0] = y2[:, :16]
        y2_ref[1] = y2[:, 16:]

    return pl.pallas_call(
        body,
        grid=(NBLK,),
        in_specs=[
            pl.BlockSpec((NCORES, BLK, 16), lambda i: (0, i, 0)),
            pl.BlockSpec((BLK, 16), lambda i: (i, 0)),
            pl.BlockSpec((BLK, 1), lambda i: (i, 0)),
            pl.BlockSpec((16, 32), lambda i: (0, 0)),
            pl.BlockSpec((1, 32), lambda i: (0, 0)),
        ],
        out_specs=pl.BlockSpec((NCORES, BLK, 16), lambda i: (0, i, 0)),
        out_shape=jax.ShapeDtypeStruct((NCORES, N_PAD, 16), jnp.float32),
    )(z1p, y1, dis, W1, b1)


def _tc_final(z2p, y2, dis, W2, b2, batch_pad, Wfc, bfc):
    def body(zp_ref, y2_ref, dis_ref, W_ref, b_ref, bt_ref, Wfc_ref, bfc_ref,
             out_ref, acc_ref):
        i = pl.program_id(0)
        dis = dis_ref[...]
        z2 = jnp.concatenate([zp_ref[0], zp_ref[1]], axis=1)
        y2 = jnp.concatenate([y2_ref[0], y2_ref[1]], axis=1)
        p = (z2 + y2) * dis
        h = jnp.dot(p, W_ref[...], preferred_element_type=jnp.float32)
        h = jnp.maximum(h + b_ref[...], 0.0)
        rows = i * BLK + lax.broadcasted_iota(jnp.int32, (BLK, 1), 0)
        valid = (rows < N).astype(jnp.float32)
        feat = jnp.concatenate([h * valid, valid], axis=1)
        onehot = (bt_ref[...] == lax.broadcasted_iota(
            jnp.int32, (BLK, G), 1)).astype(jnp.float32)
        contrib = lax.dot_general(
            onehot, feat, (((0,), (0,)), ((), ())),
            preferred_element_type=jnp.float32)

        @pl.when(i == 0)
        def _():
            acc_ref[...] = contrib

        @pl.when(i > 0)
        def _():
            acc_ref[...] = acc_ref[...] + contrib

        @pl.when(i == NBLK - 1)
        def _():
            ssum = acc_ref[:, :64]
            cnt = acc_ref[:, 64:65]
            g = ssum / jnp.maximum(cnt, 1.0)
            out_ref[...] = jnp.dot(
                g, Wfc_ref[...],
                preferred_element_type=jnp.float32) + bfc_ref[...]

    return pl.pallas_call(
        body,
        grid=(NBLK,),
        in_specs=[
            pl.BlockSpec((NCORES, BLK, 16), lambda i: (0, i, 0)),
            pl.BlockSpec((NCORES, BLK, 16), lambda i: (0, i, 0)),
            pl.BlockSpec((BLK, 1), lambda i: (i, 0)),
            pl.BlockSpec((32, 64), lambda i: (0, 0)),
            pl.BlockSpec((1, 64), lambda i: (0, 0)),
            pl.BlockSpec((BLK, 1), lambda i: (i, 0)),
            pl.BlockSpec((64, 2), lambda i: (0, 0)),
            pl.BlockSpec((1, 2), lambda i: (0, 0)),
        ],
        out_specs=pl.BlockSpec((G, 2), lambda i: (0, 0)),
        out_shape=jax.ShapeDtypeStruct((G, 2), jnp.float32),
        scratch_shapes=[pltpu.VMEM((G, 65), jnp.float32)],
    )(z2p, y2, dis, W2, b2, batch_pad, Wfc, bfc)


def kernel(x, edge_index, batch, W1, b1, W2, b2, Wfc, bfc):
    ei = edge_index.astype(jnp.int32)
    pad = jnp.arange(E_PAD - E, dtype=jnp.int32) % (N_PAD - N) + N
    src = jnp.concatenate([ei[0], pad]).reshape(E_PAD // CHUNK, CHUNK)
    dst = jnp.concatenate([ei[1], pad]).reshape(E_PAD // CHUNK, CHUNK)
    x_pad = jnp.zeros((N_PAD, 4), jnp.float32).at[:N].set(x)
    batch_pad = jnp.zeros((N_PAD, 1), jnp.int32).at[:N, 0].set(
        batch.astype(jnp.int32))
    ones = jnp.ones((CHUNK, 16), jnp.float32)
    zin16 = jnp.zeros((N_PAD, 16), jnp.float32)
    W1p = jnp.zeros((16, 32), jnp.float32).at[:4].set(W1)

    degp = _sc_degree(dst, ones, zin16)
    dis, y1 = _tc_prep(degp, x_pad)
    z1p = _sc_prop4(y1, src, dst, zin16)
    y2h = _tc_mid(z1p, y1, dis, W1p, b1.reshape(1, 32))
    z2p = _sc_prop_split(y2h, src, dst, zin16)
    return _tc_final(z2p, y2h, dis, W2, b2.reshape(1, 64), batch_pad,
                     Wfc, bfc.reshape(1, 2))


# speedup vs baseline: 65.7785x; 1.0591x over previous
import functools

import jax
import jax.numpy as jnp
from jax import lax
from jax.experimental import pallas as pl
from jax.experimental.pallas import tpu as pltpu
from jax.experimental.pallas import tpu_sc as plsc

N = 50000
G = 512
N_PAD = 50176
E = 1_600_000
CHUNK = 128
NCORES, NSUB = 2, 16
NW = NCORES * NSUB
BF = 8
PER_W = 400
E_PAD = NW * PER_W * CHUNK
NGRP_A = PER_W // BF
PER_T = NW * PER_W // NSUB
BF_S = 16
NGRP_S = PER_T // BF_S
RPT = N_PAD // NSUB
BLK = 3584
NBLK = N_PAD // BLK

_MESH = dict(core_axis_name="c", subcore_axis_name="s")
_SC_PARAMS = pltpu.CompilerParams(use_tc_tiling_on_sc=False)


def _edge_ring(ytab, src_hbm, dst_hbm, z_sh,
               sv0, sv1, dv0, dv1, r0, r1, gs0, gs1, ss0, ss1,
               base0, ngroups, bf):
    def load(sv, dv, k):
        base = base0 + k * bf
        pltpu.sync_copy(src_hbm.at[pl.ds(base, bf)], sv)
        pltpu.sync_copy(dst_hbm.at[pl.ds(base, bf)], dv)

    def fire_g(sv, rv, sem):
        for j in range(bf):
            pltpu.async_copy(ytab.at[sv.at[j]], rv.at[j], sem)

    def drain_g(sv, rv, sem):
        for j in range(bf):
            pltpu.make_async_copy(ytab.at[sv.at[j]], rv.at[j], sem).wait()

    def fire_s(dv, rv, sem):
        for j in range(bf):
            pltpu.async_copy(rv.at[j], z_sh.at[dv.at[j]], sem, add=True)

    def drain_s(dv, rv, sem):
        for j in range(bf):
            pltpu.make_async_copy(rv.at[j], z_sh.at[dv.at[j]], sem).wait()

    load(sv0, dv0, 0)
    fire_g(sv0, r0, gs0)
    npair = ngroups // 2

    def pair(t, carry):
        @pl.when(t > 0)
        def _():
            drain_s(dv1, r1, ss1)

        load(sv1, dv1, 2 * t + 1)
        drain_g(sv0, r0, gs0)
        fire_s(dv0, r0, ss0)
        fire_g(sv1, r1, gs1)
        drain_s(dv0, r0, ss0)

        @pl.when(t + 1 < npair)
        def _():
            load(sv0, dv0, 2 * t + 2)
            fire_g(sv0, r0, gs0)

        drain_g(sv1, r1, gs1)
        fire_s(dv1, r1, ss1)
        return carry

    lax.fori_loop(0, npair, pair, 0)
    drain_s(dv1, r1, ss1)


def _prop_scratch(bf):
    return [
        pltpu.VMEM((bf, CHUNK), jnp.int32),
        pltpu.VMEM((bf, CHUNK), jnp.int32),
        pltpu.VMEM((bf, CHUNK), jnp.int32),
        pltpu.VMEM((bf, CHUNK), jnp.int32),
        pltpu.VMEM((bf, CHUNK, 16), jnp.float32),
        pltpu.VMEM((bf, CHUNK, 16), jnp.float32),
        pltpu.VMEM_SHARED((N_PAD, 16), jnp.float32),
        pltpu.SemaphoreType.DMA,
        pltpu.SemaphoreType.DMA,
        pltpu.SemaphoreType.DMA,
        pltpu.SemaphoreType.DMA,
    ]


@functools.partial(
    pl.kernel,
    out_type=jax.ShapeDtypeStruct((NCORES, N_PAD, 16), jnp.float32),
    mesh=plsc.VectorSubcoreMesh(**_MESH),
    scratch_types=_prop_scratch(BF),
    compiler_params=_SC_PARAMS,
)
def _sc_prop4(y_hbm, src_hbm, dst_hbm, zinit_hbm, out_hbm,
              sv0, sv1, dv0, dv1, r0, r1, z_sh, gs0, gs1, ss0, ss1):
    c = lax.axis_index("c")
    s = lax.axis_index("s")
    w = c * NSUB + s
    pltpu.sync_copy(zinit_hbm.at[pl.ds(s * RPT, RPT)],
                    z_sh.at[pl.ds(s * RPT, RPT)])
    plsc.subcore_barrier()
    _edge_ring(y_hbm, src_hbm, dst_hbm, z_sh,
               sv0, sv1, dv0, dv1, r0, r1, gs0, gs1, ss0, ss1,
               w * PER_W, NGRP_A, BF)
    plsc.subcore_barrier()
    pltpu.sync_copy(z_sh.at[pl.ds(s * RPT, RPT)],
                    out_hbm.at[c, pl.ds(s * RPT, RPT)])


@functools.partial(
    pl.kernel,
    out_type=jax.ShapeDtypeStruct((NCORES, N_PAD, 16), jnp.float32),
    mesh=plsc.VectorSubcoreMesh(**_MESH),
    scratch_types=_prop_scratch(BF_S),
    compiler_params=_SC_PARAMS,
)
def _sc_prop_split(y_hbm, src_hbm, dst_hbm, zinit_hbm, out_hbm,
                   sv0, sv1, dv0, dv1, r0, r1, z_sh, gs0, gs1, ss0, ss1):
    c = lax.axis_index("c")
    s = lax.axis_index("s")
    pltpu.sync_copy(zinit_hbm.at[pl.ds(s * RPT, RPT)],
                    z_sh.at[pl.ds(s * RPT, RPT)])
    plsc.subcore_barrier()
    _edge_ring(y_hbm.at[c], src_hbm, dst_hbm, z_sh,
               sv0, sv1, dv0, dv1, r0, r1, gs0, gs1, ss0, ss1,
               s * PER_T, NGRP_S, BF_S)
    plsc.subcore_barrier()
    pltpu.sync_copy(z_sh.at[pl.ds(s * RPT, RPT)],
                    out_hbm.at[c, pl.ds(s * RPT, RPT)])


@functools.partial(
    pl.kernel,
    out_type=jax.ShapeDtypeStruct((NCORES, N_PAD, 16), jnp.float32),
    mesh=plsc.VectorSubcoreMesh(**_MESH),
    scratch_types=[
        pltpu.VMEM((BF, CHUNK), jnp.int32),
        pltpu.VMEM((BF, CHUNK), jnp.int32),
        pltpu.VMEM((CHUNK, 16), jnp.float32),
        pltpu.VMEM_SHARED((N_PAD, 16), jnp.float32),
        pltpu.SemaphoreType.DMA,
        pltpu.SemaphoreType.DMA,
    ],
    compiler_params=_SC_PARAMS,
)
def _sc_degree(dst_hbm, ones_hbm, zinit_hbm, out_hbm,
               dv0, dv1, ones_v, z_sh, ss0, ss1):
    c = lax.axis_index("c")
    s = lax.axis_index("s")
    w = c * NSUB + s
    base0 = w * PER_W
    pltpu.sync_copy(zinit_hbm.at[pl.ds(s * RPT, RPT)],
                    z_sh.at[pl.ds(s * RPT, RPT)])
    pltpu.sync_copy(ones_hbm, ones_v)
    plsc.subcore_barrier()

    def load(dv, k):
        pltpu.sync_copy(dst_hbm.at[pl.ds(base0 + k * BF, BF)], dv)

    def fire(dv, sem):
        for j in range(BF):
            pltpu.async_copy(ones_v, z_sh.at[dv.at[j]], sem, add=True)

    def drain(dv, sem):
        for j in range(BF):
            pltpu.make_async_copy(ones_v, z_sh.at[dv.at[j]], sem).wait()

    load(dv0, 0)
    fire(dv0, ss0)
    npair = NGRP_A // 2

    def pair(t, carry):
        @pl.when(t > 0)
        def _():
            drain(dv1, ss1)

        load(dv1, 2 * t + 1)
        fire(dv1, ss1)
        drain(dv0, ss0)

        @pl.when(t + 1 < npair)
        def _():
            load(dv0, 2 * t + 2)
            fire(dv0, ss0)

        return carry

    lax.fori_loop(0, npair, pair, 0)
    drain(dv1, ss1)
    plsc.subcore_barrier()
    pltpu.sync_copy(z_sh.at[pl.ds(s * RPT, RPT)],
                    out_hbm.at[c, pl.ds(s * RPT, RPT)])


def _tc_prep(degp, x_pad):
    def body(degp_ref, x_ref, dis_ref, y1_ref):
        deg = degp_ref[0, :, 0:1] + degp_ref[1, :, 0:1] + 1.0
        dis = lax.rsqrt(deg)
        dis_ref[...] = dis
        y1_ref[...] = jnp.concatenate(
            [x_ref[...] * dis, jnp.zeros((BLK, 12), jnp.float32)], axis=1)

    return pl.pallas_call(
        body,
        grid=(NBLK,),
        in_specs=[
            pl.BlockSpec((NCORES, BLK, 16), lambda i: (0, i, 0)),
            pl.BlockSpec((BLK, 4), lambda i: (i, 0)),
        ],
        out_specs=[
            pl.BlockSpec((BLK, 1), lambda i: (i, 0)),
            pl.BlockSpec((BLK, 16), lambda i: (i, 0)),
        ],
        out_shape=[
            jax.ShapeDtypeStruct((N_PAD, 1), jnp.float32),
            jax.ShapeDtypeStruct((N_PAD, 16), jnp.float32),
        ],
    )(degp, x_pad)


def _tc_mid(z1p, y1, dis, W1, b1):
    def body(zp_ref, y1_ref, dis_ref, W_ref, b_ref, y2_ref):
        dis = dis_ref[...]
        p = (zp_ref[0] + zp_ref[1] + y1_ref[...]) * dis
        h = jnp.dot(p, W_ref[...], preferred_element_type=jnp.float32)
        h = jnp.maximum(h + b_ref[...], 0.0)
        rows = pl.program_id(0) * BLK + lax.broadcasted_iota(
            jnp.int32, (BLK, 1), 0)
        y2 = jnp.where(rows < N, h * dis, 0.0)
        y2_ref[0] = y2[:, :16]
        y2_ref[1] = y2[:, 16:]

    return pl.pallas_call(
        body,
        grid=(NBLK,),
        in_specs=[
            pl.BlockSpec((NCORES, BLK, 16), lambda i: (0, i, 0)),
            pl.BlockSpec((BLK, 16), lambda i: (i, 0)),
            pl.BlockSpec((BLK, 1), lambda i: (i, 0)),
            pl.BlockSpec((16, 32), lambda i: (0, 0)),
            pl.BlockSpec((1, 32), lambda i: (0, 0)),
        ],
        out_specs=pl.BlockSpec((NCORES, BLK, 16), lambda i: (0, i, 0)),
        out_shape=jax.ShapeDtypeStruct((NCORES, N_PAD, 16), jnp.float32),
    )(z1p, y1, dis, W1, b1)


def _tc_final(z2p, y2, dis, W2, b2, batch_pad, Wfc, bfc):
    def body(zp_ref, y2_ref, dis_ref, W_ref, b_ref, bt_ref, Wfc_ref, bfc_ref,
             out_ref, acc_ref):
        i = pl.program_id(0)
        dis = dis_ref[...]
        z2 = jnp.concatenate([zp_ref[0], zp_ref[1]], axis=1)
        y2 = jnp.concatenate([y2_ref[0], y2_ref[1]], axis=1)
        p = (z2 + y2) * dis
        h = jnp.dot(p, W_ref[...], preferred_element_type=jnp.float32)
        h = jnp.maximum(h + b_ref[...], 0.0)
        rows = i * BLK + lax.broadcasted_iota(jnp.int32, (BLK, 1), 0)
        valid = (rows < N).astype(jnp.float32)
        feat = jnp.concatenate([h * valid, valid], axis=1)
        onehot = (bt_ref[...] == lax.broadcasted_iota(
            jnp.int32, (BLK, G), 1)).astype(jnp.float32)
        contrib = lax.dot_general(
            onehot, feat, (((0,), (0,)), ((), ())),
            preferred_element_type=jnp.float32)

        @pl.when(i == 0)
        def _():
            acc_ref[...] = contrib

        @pl.when(i > 0)
        def _():
            acc_ref[...] = acc_ref[...] + contrib

        @pl.when(i == NBLK - 1)
        def _():
            ssum = acc_ref[:, :64]
            cnt = acc_ref[:, 64:65]
            g = ssum / jnp.maximum(cnt, 1.0)
            out_ref[...] = jnp.dot(
                g, Wfc_ref[...],
                preferred_element_type=jnp.float32) + bfc_ref[...]

    return pl.pallas_call(
        body,
        grid=(NBLK,),
        in_specs=[
            pl.BlockSpec((NCORES, BLK, 16), lambda i: (0, i, 0)),
            pl.BlockSpec((NCORES, BLK, 16), lambda i: (0, i, 0)),
            pl.BlockSpec((BLK, 1), lambda i: (i, 0)),
            pl.BlockSpec((32, 64), lambda i: (0, 0)),
            pl.BlockSpec((1, 64), lambda i: (0, 0)),
            pl.BlockSpec((BLK, 1), lambda i: (i, 0)),
            pl.BlockSpec((64, 2), lambda i: (0, 0)),
            pl.BlockSpec((1, 2), lambda i: (0, 0)),
        ],
        out_specs=pl.BlockSpec((G, 2), lambda i: (0, 0)),
        out_shape=jax.ShapeDtypeStruct((G, 2), jnp.float32),
        scratch_shapes=[pltpu.VMEM((G, 65), jnp.float32)],
    )(z2p, y2, dis, W2, b2, batch_pad, Wfc, bfc)


def kernel(x, edge_index, batch, W1, b1, W2, b2, Wfc, bfc):
    ei = edge_index.astype(jnp.int32)
    pad = jnp.arange(E_PAD - E, dtype=jnp.int32) % (N_PAD - N) + N
    src = jnp.concatenate([ei[0], pad]).reshape(E_PAD // CHUNK, CHUNK)
    dst = jnp.concatenate([ei[1], pad]).reshape(E_PAD // CHUNK, CHUNK)
    x_pad = jnp.zeros((N_PAD, 4), jnp.float32).at[:N].set(x)
    batch_pad = jnp.zeros((N_PAD, 1), jnp.int32).at[:N, 0].set(
        batch.astype(jnp.int32))
    ones = jnp.ones((CHUNK, 16), jnp.float32)
    zin16 = jnp.zeros((N_PAD, 16), jnp.float32)
    W1p = jnp.zeros((16, 32), jnp.float32).at[:4].set(W1)

    degp = _sc_degree(dst, ones, zin16)
    dis, y1 = _tc_prep(degp, x_pad)
    z1p = _sc_prop4(y1, src, dst, zin16)
    y2h = _tc_mid(z1p, y1, dis, W1p, b1.reshape(1, 32))
    z2p = _sc_prop_split(y2h, src, dst, zin16)
    return _tc_final(z2p, y2h, dis, W2, b2.reshape(1, 64), batch_pad,
                     Wfc, bfc.reshape(1, 2))


# speedup vs baseline: 66.2546x; 1.0072x over previous
import functools

import jax
import jax.numpy as jnp
from jax import lax
from jax.experimental import pallas as pl
from jax.experimental.pallas import tpu as pltpu
from jax.experimental.pallas import tpu_sc as plsc

N = 50000
G = 512
N_PAD = 50176
E = 1_600_000
CHUNK = 128
NCHUNK = E // CHUNK
NCORES, NSUB = 2, 16
NW = NCORES * NSUB
BF = 8
BF_S = 16
MAIN_A = 781 * 2 * BF
NPAIR_A, XTRA_A = 24, 13
NPAIR_S, XTRA_S = 24, 6
SPLIT_LEFT = 390 * 2 * BF_S
TAIL = NCHUNK - MAIN_A
RPT = N_PAD // NSUB
BLK = 3584
NBLK = N_PAD // BLK

_MESH = dict(core_axis_name="c", subcore_axis_name="s")
_SC_PARAMS = pltpu.CompilerParams(use_tc_tiling_on_sc=False)


def _edge_ring(ytab, src_hbm, dst_hbm, z_sh,
               sv0, sv1, dv0, dv1, r0, r1, gs0, gs1, ss0, ss1,
               base0, npair, bf):
    def load(sv, dv, k):
        base = base0 + k * bf
        pltpu.sync_copy(src_hbm.at[pl.ds(base, bf)], sv)
        pltpu.sync_copy(dst_hbm.at[pl.ds(base, bf)], dv)

    def fire_g(sv, rv, sem):
        for j in range(bf):
            pltpu.async_copy(ytab.at[sv.at[j]], rv.at[j], sem)

    def drain_g(sv, rv, sem):
        for j in range(bf):
            pltpu.make_async_copy(ytab.at[sv.at[j]], rv.at[j], sem).wait()

    def fire_s(dv, rv, sem):
        for j in range(bf):
            pltpu.async_copy(rv.at[j], z_sh.at[dv.at[j]], sem, add=True)

    def drain_s(dv, rv, sem):
        for j in range(bf):
            pltpu.make_async_copy(rv.at[j], z_sh.at[dv.at[j]], sem).wait()

    load(sv0, dv0, 0)
    fire_g(sv0, r0, gs0)

    def pair(t, carry):
        @pl.when(t > 0)
        def _():
            drain_s(dv1, r1, ss1)

        load(sv1, dv1, 2 * t + 1)
        drain_g(sv0, r0, gs0)
        fire_s(dv0, r0, ss0)
        fire_g(sv1, r1, gs1)
        drain_s(dv0, r0, ss0)

        @pl.when(t + 1 < npair)
        def _():
            load(sv0, dv0, 2 * t + 2)
            fire_g(sv0, r0, gs0)

        drain_g(sv1, r1, gs1)
        fire_s(dv1, r1, ss1)
        return carry

    lax.fori_loop(0, npair, pair, 0)
    drain_s(dv1, r1, ss1)


def _tail_prop(ytab, tsrc_hbm, tdst_hbm, z_sh, sv, dv, rv, gsem, ssem, nrow):
    pltpu.sync_copy(tsrc_hbm, sv.at[pl.ds(0, nrow)])
    pltpu.sync_copy(tdst_hbm, dv.at[pl.ds(0, nrow)])
    for j in range(nrow):
        pltpu.async_copy(ytab.at[sv.at[j]], rv.at[j], gsem)
    for j in range(nrow):
        pltpu.make_async_copy(ytab.at[sv.at[j]], rv.at[j], gsem).wait()
    for j in range(nrow):
        pltpu.async_copy(rv.at[j], z_sh.at[dv.at[j]], ssem, add=True)
    for j in range(nrow):
        pltpu.make_async_copy(rv.at[j], z_sh.at[dv.at[j]], ssem).wait()


def _prop_scratch(bf):
    return [
        pltpu.VMEM((bf, CHUNK), jnp.int32),
        pltpu.VMEM((bf, CHUNK), jnp.int32),
        pltpu.VMEM((bf, CHUNK), jnp.int32),
        pltpu.VMEM((bf, CHUNK), jnp.int32),
        pltpu.VMEM((bf, CHUNK, 16), jnp.float32),
        pltpu.VMEM((bf, CHUNK, 16), jnp.float32),
        pltpu.VMEM_SHARED((N_PAD, 16), jnp.float32),
        pltpu.SemaphoreType.DMA,
        pltpu.SemaphoreType.DMA,
        pltpu.SemaphoreType.DMA,
        pltpu.SemaphoreType.DMA,
    ]


@functools.partial(
    pl.kernel,
    out_type=jax.ShapeDtypeStruct((NCORES, N_PAD, 16), jnp.float32),
    mesh=plsc.VectorSubcoreMesh(**_MESH),
    scratch_types=_prop_scratch(BF),
    compiler_params=_SC_PARAMS,
)
def _sc_prop4(y_hbm, src_hbm, dst_hbm, tsrc_hbm, tdst_hbm, zinit_hbm, out_hbm,
              sv0, sv1, dv0, dv1, r0, r1, z_sh, gs0, gs1, ss0, ss1):
    c = lax.axis_index("c")
    s = lax.axis_index("s")
    w = c * NSUB + s
    pltpu.sync_copy(zinit_hbm.at[pl.ds(s * RPT, RPT)],
                    z_sh.at[pl.ds(s * RPT, RPT)])
    plsc.subcore_barrier()
    npair = jnp.where(w < XTRA_A, NPAIR_A + 1, NPAIR_A)
    base0 = 2 * BF * (NPAIR_A * w + jnp.minimum(w, XTRA_A))
    _edge_ring(y_hbm, src_hbm, dst_hbm, z_sh,
               sv0, sv1, dv0, dv1, r0, r1, gs0, gs1, ss0, ss1,
               base0, npair, BF)

    @pl.when(w == NW - 1)
    def _():
        _tail_prop(y_hbm, tsrc_hbm, tdst_hbm, z_sh,
                   sv0, dv0, r0, gs0, ss0, TAIL)

    plsc.subcore_barrier()
    pltpu.sync_copy(z_sh.at[pl.ds(s * RPT, RPT)],
                    out_hbm.at[c, pl.ds(s * RPT, RPT)])


@functools.partial(
    pl.kernel,
    out_type=jax.ShapeDtypeStruct((NCORES, N_PAD, 16), jnp.float32),
    mesh=plsc.VectorSubcoreMesh(**_MESH),
    scratch_types=_prop_scratch(BF_S),
    compiler_params=_SC_PARAMS,
)
def _sc_prop_split(y_hbm, src_hbm, dst_hbm, tsrc_hbm, tdst_hbm, zinit_hbm,
                   out_hbm,
                   sv0, sv1, dv0, dv1, r0, r1, z_sh, gs0, gs1, ss0, ss1):
    c = lax.axis_index("c")
    s = lax.axis_index("s")
    pltpu.sync_copy(zinit_hbm.at[pl.ds(s * RPT, RPT)],
                    z_sh.at[pl.ds(s * RPT, RPT)])
    plsc.subcore_barrier()
    npair = jnp.where(s < XTRA_S, NPAIR_S + 1, NPAIR_S)
    base0 = 2 * BF_S * (NPAIR_S * s + jnp.minimum(s, XTRA_S))
    _edge_ring(y_hbm.at[c], src_hbm, dst_hbm, z_sh,
               sv0, sv1, dv0, dv1, r0, r1, gs0, gs1, ss0, ss1,
               base0, npair, BF_S)

    @pl.when(s == NSUB - 1)
    def _():
        pltpu.sync_copy(src_hbm.at[pl.ds(SPLIT_LEFT, BF_S)], sv1)
        pltpu.sync_copy(dst_hbm.at[pl.ds(SPLIT_LEFT, BF_S)], dv1)
        for j in range(BF_S):
            pltpu.async_copy(y_hbm.at[c].at[sv1.at[j]], r1.at[j], gs1)
        for j in range(BF_S):
            pltpu.make_async_copy(y_hbm.at[c].at[sv1.at[j]], r1.at[j],
                                  gs1).wait()
        for j in range(BF_S):
            pltpu.async_copy(r1.at[j], z_sh.at[dv1.at[j]], ss1, add=True)
        for j in range(BF_S):
            pltpu.make_async_copy(r1.at[j], z_sh.at[dv1.at[j]], ss1).wait()
        _tail_prop(y_hbm.at[c], tsrc_hbm, tdst_hbm, z_sh,
                   sv0, dv0, r0, gs0, ss0, TAIL)

    plsc.subcore_barrier()
    pltpu.sync_copy(z_sh.at[pl.ds(s * RPT, RPT)],
                    out_hbm.at[c, pl.ds(s * RPT, RPT)])


@functools.partial(
    pl.kernel,
    out_type=jax.ShapeDtypeStruct((NCORES, N_PAD, 16), jnp.float32),
    mesh=plsc.VectorSubcoreMesh(**_MESH),
    scratch_types=[
        pltpu.VMEM((BF, CHUNK), jnp.int32),
        pltpu.VMEM((BF, CHUNK), jnp.int32),
        pltpu.VMEM((CHUNK, 16), jnp.float32),
        pltpu.VMEM_SHARED((N_PAD, 16), jnp.float32),
        pltpu.SemaphoreType.DMA,
        pltpu.SemaphoreType.DMA,
    ],
    compiler_params=_SC_PARAMS,
)
def _sc_degree(dst_hbm, tdst_hbm, ones_hbm, zinit_hbm, out_hbm,
               dv0, dv1, ones_v, z_sh, ss0, ss1):
    c = lax.axis_index("c")
    s = lax.axis_index("s")
    w = c * NSUB + s
    npair = jnp.where(w < XTRA_A, NPAIR_A + 1, NPAIR_A)
    base0 = 2 * BF * (NPAIR_A * w + jnp.minimum(w, XTRA_A))
    pltpu.sync_copy(zinit_hbm.at[pl.ds(s * RPT, RPT)],
                    z_sh.at[pl.ds(s * RPT, RPT)])
    pltpu.sync_copy(ones_hbm, ones_v)
    plsc.subcore_barrier()

    def load(dv, k):
        pltpu.sync_copy(dst_hbm.at[pl.ds(base0 + k * BF, BF)], dv)

    def fire(dv, sem):
        for j in range(BF):
            pltpu.async_copy(ones_v, z_sh.at[dv.at[j]], sem, add=True)

    def drain(dv, sem):
        for j in range(BF):
            pltpu.make_async_copy(ones_v, z_sh.at[dv.at[j]], sem).wait()

    load(dv0, 0)
    fire(dv0, ss0)

    def pair(t, carry):
        @pl.when(t > 0)
        def _():
            drain(dv1, ss1)

        load(dv1, 2 * t + 1)
        fire(dv1, ss1)
        drain(dv0, ss0)

        @pl.when(t + 1 < npair)
        def _():
            load(dv0, 2 * t + 2)
            fire(dv0, ss0)

        return carry

    lax.fori_loop(0, npair, pair, 0)
    drain(dv1, ss1)

    @pl.when(w == NW - 1)
    def _():
        pltpu.sync_copy(tdst_hbm, dv0.at[pl.ds(0, TAIL)])
        for j in range(TAIL):
            pltpu.async_copy(ones_v, z_sh.at[dv0.at[j]], ss0, add=True)
        for j in range(TAIL):
            pltpu.make_async_copy(ones_v, z_sh.at[dv0.at[j]], ss0).wait()

    plsc.subcore_barrier()
    pltpu.sync_copy(z_sh.at[pl.ds(s * RPT, RPT)],
                    out_hbm.at[c, pl.ds(s * RPT, RPT)])


def _tc_prep(degp, x_pad):
    def body(degp_ref, x_ref, dis_ref, y1_ref):
        deg = degp_ref[0, :, 0:1] + degp_ref[1, :, 0:1] + 1.0
        dis = lax.rsqrt(deg)
        dis_ref[...] = dis
        y1_ref[...] = jnp.concatenate(
            [x_ref[...] * dis, jnp.zeros((BLK, 12), jnp.float32)], axis=1)

    return pl.pallas_call(
        body,
        grid=(NBLK,),
        in_specs=[
            pl.BlockSpec((NCORES, BLK, 16), lambda i: (0, i, 0)),
            pl.BlockSpec((BLK, 4), lambda i: (i, 0)),
        ],
        out_specs=[
            pl.BlockSpec((BLK, 1), lambda i: (i, 0)),
            pl.BlockSpec((BLK, 16), lambda i: (i, 0)),
        ],
        out_shape=[
            jax.ShapeDtypeStruct((N_PAD, 1), jnp.float32),
            jax.ShapeDtypeStruct((N_PAD, 16), jnp.float32),
        ],
    )(degp, x_pad)


def _tc_mid(z1p, y1, dis, W1, b1):
    def body(zp_ref, y1_ref, dis_ref, W_ref, b_ref, y2_ref):
        dis = dis_ref[...]
        p = (zp_ref[0] + zp_ref[1] + y1_ref[...]) * dis
        h = jnp.dot(p, W_ref[...], preferred_element_type=jnp.float32)
        h = jnp.maximum(h + b_ref[...], 0.0)
        rows = pl.program_id(0) * BLK + lax.broadcasted_iota(
            jnp.int32, (BLK, 1), 0)
        y2 = jnp.where(rows < N, h * dis, 0.0)
        y2_ref[0] = y2[:, :16]
        y2_ref[1] = y2[:, 16:]

    return pl.pallas_call(
        body,
        grid=(NBLK,),
        in_specs=[
            pl.BlockSpec((NCORES, BLK, 16), lambda i: (0, i, 0)),
            pl.BlockSpec((BLK, 16), lambda i: (i, 0)),
            pl.BlockSpec((BLK, 1), lambda i: (i, 0)),
            pl.BlockSpec((16, 32), lambda i: (0, 0)),
            pl.BlockSpec((1, 32), lambda i: (0, 0)),
        ],
        out_specs=pl.BlockSpec((NCORES, BLK, 16), lambda i: (0, i, 0)),
        out_shape=jax.ShapeDtypeStruct((NCORES, N_PAD, 16), jnp.float32),
    )(z1p, y1, dis, W1, b1)


def _tc_final(z2p, y2, dis, W2, b2, batch_pad, Wfc, bfc):
    def body(zp_ref, y2_ref, dis_ref, W_ref, b_ref, bt_ref, Wfc_ref, bfc_ref,
             out_ref, acc_ref):
        i = pl.program_id(0)
        dis = dis_ref[...]
        z2 = jnp.concatenate([zp_ref[0], zp_ref[1]], axis=1)
        y2 = jnp.concatenate([y2_ref[0], y2_ref[1]], axis=1)
        p = (z2 + y2) * dis
        h = jnp.dot(p, W_ref[...], preferred_element_type=jnp.float32)
        h = jnp.maximum(h + b_ref[...], 0.0)
        rows = i * BLK + lax.broadcasted_iota(jnp.int32, (BLK, 1), 0)
        valid = (rows < N).astype(jnp.float32)
        feat = jnp.concatenate([h * valid, valid], axis=1)
        onehot = (bt_ref[...] == lax.broadcasted_iota(
            jnp.int32, (BLK, G), 1)).astype(jnp.float32)
        contrib = lax.dot_general(
            onehot, feat, (((0,), (0,)), ((), ())),
            preferred_element_type=jnp.float32)

        @pl.when(i == 0)
        def _():
            acc_ref[...] = contrib

        @pl.when(i > 0)
        def _():
            acc_ref[...] = acc_ref[...] + contrib

        @pl.when(i == NBLK - 1)
        def _():
            ssum = acc_ref[:, :64]
            cnt = acc_ref[:, 64:65]
            g = ssum / jnp.maximum(cnt, 1.0)
            out_ref[...] = jnp.dot(
                g, Wfc_ref[...],
                preferred_element_type=jnp.float32) + bfc_ref[...]

    return pl.pallas_call(
        body,
        grid=(NBLK,),
        in_specs=[
            pl.BlockSpec((NCORES, BLK, 16), lambda i: (0, i, 0)),
            pl.BlockSpec((NCORES, BLK, 16), lambda i: (0, i, 0)),
            pl.BlockSpec((BLK, 1), lambda i: (i, 0)),
            pl.BlockSpec((32, 64), lambda i: (0, 0)),
            pl.BlockSpec((1, 64), lambda i: (0, 0)),
            pl.BlockSpec((BLK, 1), lambda i: (i, 0)),
            pl.BlockSpec((64, 2), lambda i: (0, 0)),
            pl.BlockSpec((1, 2), lambda i: (0, 0)),
        ],
        out_specs=pl.BlockSpec((G, 2), lambda i: (0, 0)),
        out_shape=jax.ShapeDtypeStruct((G, 2), jnp.float32),
        scratch_shapes=[pltpu.VMEM((G, 65), jnp.float32)],
    )(z2p, y2, dis, W2, b2, batch_pad, Wfc, bfc)


def kernel(x, edge_index, batch, W1, b1, W2, b2, Wfc, bfc):
    ei = edge_index.astype(jnp.int32)
    src = ei[0].reshape(NCHUNK, CHUNK)
    dst = ei[1].reshape(NCHUNK, CHUNK)
    tsrc = src[MAIN_A:]
    tdst = dst[MAIN_A:]
    x_pad = jnp.zeros((N_PAD, 4), jnp.float32).at[:N].set(x)
    batch_pad = jnp.zeros((N_PAD, 1), jnp.int32).at[:N, 0].set(
        batch.astype(jnp.int32))
    ones = jnp.ones((CHUNK, 16), jnp.float32)
    zin16 = jnp.zeros((N_PAD, 16), jnp.float32)
    W1p = jnp.zeros((16, 32), jnp.float32).at[:4].set(W1)

    degp = _sc_degree(dst, tdst, ones, zin16)
    dis, y1 = _tc_prep(degp, x_pad)
    z1p = _sc_prop4(y1, src, dst, tsrc, tdst, zin16)
    y2h = _tc_mid(z1p, y1, dis, W1p, b1.reshape(1, 32))
    z2p = _sc_prop_split(y2h, src, dst, tsrc, tdst, zin16)
    return _tc_final(z2p, y2h, dis, W2, b2.reshape(1, 64), batch_pad,
                     Wfc, bfc.reshape(1, 2))
